# Initial kernel scaffold; baseline (speedup 1.0000x reference)
import jax
import jax.numpy as jnp
from jax.experimental import pallas as pl


def kernel(x, Wg, W_gate, W_up, W_down, Ws_gate, Ws_up, Ws_down):
    raise NotImplementedError("write your pallas kernel here")


# speedup vs baseline: 1.8354x; 1.8354x over previous
import jax
import jax.numpy as jnp
from jax.experimental import pallas as pl
from jax.experimental.pallas import tpu as pltpu

_B, _C, _H, _W = 4, 384, 32, 32
_E = 8
_FF = 768
_N = _B * _H * _W
_TB = 1024
_NTB = _N // _TB


def _top2_weight_for_expert(probs, e):
    lane = jax.lax.broadcasted_iota(jnp.int32, probs.shape, 1)
    m1 = jnp.max(probs, axis=1, keepdims=True)
    i1 = jnp.min(jnp.where(probs == m1, lane, _E), axis=1, keepdims=True)
    p2 = jnp.where(lane == i1, -1e30, probs)
    m2 = jnp.max(p2, axis=1, keepdims=True)
    i2 = jnp.min(jnp.where(p2 == m2, lane, _E), axis=1, keepdims=True)
    s = m1 + m2
    w1 = (m1 / s)[:, 0]
    w2 = (m2 / s)[:, 0]
    return jnp.where(i1[:, 0] == e, w1, jnp.where(i2[:, 0] == e, w2, 0.0))


def _dense_body(wg_ref, x_ref, weg_ref, weu_ref, wed_ref,
                wsg_ref, wsu_ref, wsd_ref, out_ref):
    e = pl.program_id(1)
    x = x_ref[...]
    logits = jnp.dot(x, wg_ref[...], preferred_element_type=jnp.float32)
    probs = jax.nn.softmax(logits, axis=-1)
    we = _top2_weight_for_expert(probs, e)
    g = jnp.dot(x, weg_ref[0], preferred_element_type=jnp.float32)
    u = jnp.dot(x, weu_ref[0], preferred_element_type=jnp.float32)
    h = g * jax.nn.sigmoid(g) * u
    o = jnp.dot(h, wed_ref[0], preferred_element_type=jnp.float32)
    o = o * we[:, None]

    @pl.when(e == 0)
    def _():
        sg = jnp.dot(x, wsg_ref[...], preferred_element_type=jnp.float32)
        su = jnp.dot(x, wsu_ref[...], preferred_element_type=jnp.float32)
        sh = jnp.dot(sg * jax.nn.sigmoid(sg) * su, wsd_ref[...],
                     preferred_element_type=jnp.float32)
        out_ref[...] = sh + o

    @pl.when(e > 0)
    def _():
        out_ref[...] += o


def _moe_dense(t, Wg, W_gate, W_up, W_down, Ws_gate, Ws_up, Ws_down):
    return pl.pallas_call(
        _dense_body,
        grid=(_NTB, _E),
        in_specs=[
            pl.BlockSpec((_C, _E), lambda tb, e: (0, 0)),
            pl.BlockSpec((_TB, _C), lambda tb, e: (tb, 0)),
            pl.BlockSpec((1, _C, _FF), lambda tb, e: (e, 0, 0)),
            pl.BlockSpec((1, _C, _FF), lambda tb, e: (e, 0, 0)),
            pl.BlockSpec((1, _FF, _C), lambda tb, e: (e, 0, 0)),
            pl.BlockSpec((_C, _FF), lambda tb, e: (0, 0)),
            pl.BlockSpec((_C, _FF), lambda tb, e: (0, 0)),
            pl.BlockSpec((_FF, _C), lambda tb, e: (0, 0)),
        ],
        out_specs=pl.BlockSpec((_TB, _C), lambda tb, e: (tb, 0)),
        out_shape=jax.ShapeDtypeStruct((_N, _C), jnp.float32),
    )(Wg, t, W_gate, W_up, W_down, Ws_gate, Ws_up, Ws_down)


def kernel(x, Wg, W_gate, W_up, W_down, Ws_gate, Ws_up, Ws_down):
    b, c, h, w = x.shape
    t = jnp.transpose(x, (0, 2, 3, 1)).reshape(b * h * w, c)
    y = _moe_dense(t, Wg, W_gate, W_up, W_down, Ws_gate, Ws_up, Ws_down)
    return jnp.transpose(y.reshape(b, h, w, c), (0, 3, 1, 2))
